# baseline (speedup 1.0000x reference)
import functools

import jax
import jax.numpy as jnp
from jax import lax
from jax.experimental import pallas as pl
from jax.experimental.pallas import tpu as pltpu
from jax.experimental.pallas import tpu_sc as plsc

B = 16384
D = 8
NF = 4
V = 1000
NW = 32
BPW = B // NW
L = 16
ENC_R = 40

TAB0 = B * NF
X0 = TAB0 + NF * V * D
NIN = X0 + B * 2

_SC_MESH = plsc.VectorSubcoreMesh(core_axis_name="c", subcore_axis_name="s")


@functools.partial(
    pl.kernel,
    out_type=jax.ShapeDtypeStruct((ENC_R, B), jnp.float32),
    mesh=_SC_MESH,
    scratch_types=[
        pltpu.VMEM((NF * V * D,), jnp.int32),
        pltpu.VMEM((NF * BPW,), jnp.int32),
        pltpu.VMEM((2 * BPW,), jnp.int32),
        pltpu.VMEM((ENC_R, BPW), jnp.float32),
        pltpu.SemaphoreType.DMA,
    ],
    compiler_params=pltpu.CompilerParams(needs_layout_passes=False),
)
def _sc_encode(allin, enc_out, tab_v, idx_v, x_v, enc_v, sem):
    wid = lax.axis_index("s") * 2 + lax.axis_index("c")
    base = wid * BPW
    copies = [
        pltpu.async_copy(allin.at[pl.ds(TAB0, NF * V * D)], tab_v, sem),
        pltpu.async_copy(allin.at[pl.ds(base * NF, NF * BPW)], idx_v, sem),
        pltpu.async_copy(allin.at[pl.ds(X0 + base * 2, 2 * BPW)], x_v, sem),
    ]
    for cp in copies:
        cp.wait()

    lane = jnp.arange(L, dtype=jnp.int32)
    lane4 = lane * NF
    lane2 = lane * 2
    zeros = jnp.zeros((L,), jnp.float32)

    @pl.loop(0, BPW // L)
    def _(g):
        s = g * L
        s4 = s * NF
        for f in range(NF):
            idxs = plsc.load_gather(idx_v, [s4 + lane4 + f])
            idx8 = idxs * D
            for j in range(D):
                v = plsc.load_gather(tab_v, [idx8 + (f * V * D + j)])
                enc_v[f * D + j, pl.ds(s, L)] = plsc.bitcast(v, jnp.float32)
        sl2 = s * 2 + lane2
        for j in range(2):
            v = plsc.load_gather(x_v, [sl2 + j])
            enc_v[NF * D + j, pl.ds(s, L)] = plsc.bitcast(v, jnp.float32)
        for r in range(NF * D + 2, ENC_R):
            enc_v[r, pl.ds(s, L)] = zeros

    pltpu.sync_copy(enc_v, enc_out.at[:, pl.ds(base, BPW)])


BM = 2048


def _mlp_body(enc_ref, w1_ref, b1_ref, w2_ref, b2_ref, w3_ref, b3_ref,
              o_ref):
    h = jnp.dot(w1_ref[...], enc_ref[...], preferred_element_type=jnp.float32)
    h = jnp.maximum(h + b1_ref[...], 0.0)
    h = jnp.dot(w2_ref[...], h, preferred_element_type=jnp.float32)
    h = jnp.maximum(h + b2_ref[...], 0.0)
    o_ref[...] = (jnp.dot(w3_ref[...], h, preferred_element_type=jnp.float32)
                  + b3_ref[...])


_mlp = pl.pallas_call(
    _mlp_body,
    grid=(B // BM,),
    in_specs=[
        pl.BlockSpec((ENC_R, BM), lambda i: (0, i)),
        pl.BlockSpec((ENC_R, ENC_R), lambda i: (0, 0)),
        pl.BlockSpec((ENC_R, 1), lambda i: (0, 0)),
        pl.BlockSpec((ENC_R, ENC_R), lambda i: (0, 0)),
        pl.BlockSpec((ENC_R, 1), lambda i: (0, 0)),
        pl.BlockSpec((2, ENC_R), lambda i: (0, 0)),
        pl.BlockSpec((2, 1), lambda i: (0, 0)),
    ],
    out_specs=pl.BlockSpec((2, BM), lambda i: (0, i)),
    out_shape=jax.ShapeDtypeStruct((2, B), jnp.float32),
)


def kernel(x, x_classes, emb0, emb1, emb2, emb3, W1, b1, W2, b2, W3, b3):
    allin = jnp.concatenate([
        x_classes.astype(jnp.int32).reshape(-1),
        lax.bitcast_convert_type(emb0, jnp.int32).reshape(-1),
        lax.bitcast_convert_type(emb1, jnp.int32).reshape(-1),
        lax.bitcast_convert_type(emb2, jnp.int32).reshape(-1),
        lax.bitcast_convert_type(emb3, jnp.int32).reshape(-1),
        lax.bitcast_convert_type(x, jnp.int32).reshape(-1),
    ])
    enc = _sc_encode(allin)
    w1r = jnp.concatenate(
        [W1[2:], W1[:2], jnp.zeros((ENC_R - 34, 40), jnp.float32)], axis=0)
    yt = _mlp(enc, w1r.T, b1.reshape(-1, 1), W2.T, b2.reshape(-1, 1),
              W3.T, b3.reshape(-1, 1))
    return yt.T


# speedup vs baseline: 1.9616x; 1.9616x over previous
import functools

import jax
import jax.numpy as jnp
from jax import lax
from jax.experimental import pallas as pl
from jax.experimental.pallas import tpu as pltpu
from jax.experimental.pallas import tpu_sc as plsc

B = 16384
D = 8
NF = 4
V = 1000
NW = 32
BPW = B // NW
L = 16
ENC_R = NF * D

_SC_MESH = plsc.VectorSubcoreMesh(core_axis_name="c", subcore_axis_name="s")


@functools.partial(
    pl.kernel,
    out_type=jax.ShapeDtypeStruct((ENC_R, B), jnp.float32),
    mesh=_SC_MESH,
    scratch_types=[
        pltpu.VMEM((NF * V * D,), jnp.float32),
        pltpu.VMEM((NF, BPW), jnp.int32),
        pltpu.VMEM((ENC_R, BPW), jnp.float32),
        pltpu.SemaphoreType.DMA,
    ],
    compiler_params=pltpu.CompilerParams(needs_layout_passes=False),
)
def _sc_encode(xc_t, e0, e1, e2, e3, enc_out, tab_v, idx_v, enc_v, sem):
    wid = lax.axis_index("s") * 2 + lax.axis_index("c")
    base = wid * BPW
    tables = (e0, e1, e2, e3)
    copies = [
        pltpu.async_copy(tables[f], tab_v.at[pl.ds(f * V * D, V * D)], sem)
        for f in range(NF)
    ]
    copies.append(pltpu.async_copy(xc_t.at[:, pl.ds(base, BPW)], idx_v, sem))
    for cp in copies:
        cp.wait()

    @functools.partial(plsc.parallel_loop, 0, BPW // L, unroll=2)
    def _(g):
        s = g * L
        for f in range(NF):
            idx8 = idx_v[f, pl.ds(s, L)] * D
            for j in range(D):
                enc_v[f * D + j, pl.ds(s, L)] = plsc.load_gather(
                    tab_v, [idx8 + (f * V * D + j)])

    pltpu.sync_copy(enc_v, enc_out.at[:, pl.ds(base, BPW)])


BM = 2048


def _mlp_body(enc_ref, xt_ref, w1e_ref, w1x_ref, b1_ref, w2_ref, b2_ref,
              w3_ref, b3_ref, o_ref):
    h = jnp.dot(w1e_ref[...], enc_ref[...], preferred_element_type=jnp.float32)
    h = h + jnp.dot(w1x_ref[...], xt_ref[...],
                    preferred_element_type=jnp.float32)
    h = jnp.maximum(h + b1_ref[...], 0.0)
    h = jnp.dot(w2_ref[...], h, preferred_element_type=jnp.float32)
    h = jnp.maximum(h + b2_ref[...], 0.0)
    o_ref[...] = (jnp.dot(w3_ref[...], h, preferred_element_type=jnp.float32)
                  + b3_ref[...])


_mlp = pl.pallas_call(
    _mlp_body,
    grid=(B // BM,),
    in_specs=[
        pl.BlockSpec((ENC_R, BM), lambda i: (0, i)),
        pl.BlockSpec((2, BM), lambda i: (0, i)),
        pl.BlockSpec((40, ENC_R), lambda i: (0, 0)),
        pl.BlockSpec((40, 2), lambda i: (0, 0)),
        pl.BlockSpec((40, 1), lambda i: (0, 0)),
        pl.BlockSpec((40, 40), lambda i: (0, 0)),
        pl.BlockSpec((40, 1), lambda i: (0, 0)),
        pl.BlockSpec((2, 40), lambda i: (0, 0)),
        pl.BlockSpec((2, 1), lambda i: (0, 0)),
    ],
    out_specs=pl.BlockSpec((2, BM), lambda i: (0, i)),
    out_shape=jax.ShapeDtypeStruct((2, B), jnp.float32),
)


def kernel(x, x_classes, emb0, emb1, emb2, emb3, W1, b1, W2, b2, W3, b3):
    xc_t = jnp.transpose(x_classes).astype(jnp.int32)
    enc = _sc_encode(xc_t, emb0.reshape(-1), emb1.reshape(-1),
                     emb2.reshape(-1), emb3.reshape(-1))
    yt = _mlp(enc, x.T, W1[2:].T, W1[:2].T, b1.reshape(-1, 1), W2.T,
              b2.reshape(-1, 1), W3.T, b3.reshape(-1, 1))
    return yt.T
